# baseline (speedup 1.0000x reference)
from math import sqrt

import jax
import jax.numpy as jnp
from jax.experimental import pallas as pl
from jax.experimental.pallas import tpu as pltpu

_B, _T, _N = 8, 96, 2000
_H = 32
_TOPK = 20
_TAU = 0.5
_ALPHA = 0.15
_NP = 2048
_R = 1024
_KBIN = 10

_PREC = jax.lax.Precision.DEFAULT
_DN_T = (((1,), (0,)), ((), ()))
_DN_H = (((0,), (0,)), ((), ()))
_DN_J = (((1,), (1,)), ((), ()))


def _proj_norm(w, x):
    p = jax.lax.dot_general(w, x, _DN_T, preferred_element_type=jnp.float32,
                            precision=_PREC)
    n = jnp.sqrt(jnp.sum(p * p, axis=0, keepdims=True))
    return p / jnp.maximum(n, 1e-12)


def _body(x_ref, wq_ref, wk_ref, o_ref, kt_ref, xm_ref):
    i = pl.program_id(1)

    @pl.when(i == 0)
    def _():
        colx = jax.lax.broadcasted_iota(jnp.int32, (_T, _NP), 1)
        xm = jnp.where(colx < _N, x_ref[0], 0.0)
        xm_ref[0:_T] = xm
        xm_ref[_T:] = jnp.ones((8, _NP), jnp.float32)
        kt_ref[...] = _proj_norm(wk_ref[...], xm)

    xa = xm_ref[...]
    x = xa[0:_T]

    xq = x_ref[0, :, pl.ds(i * _R, _R)]
    qt = _proj_norm(wq_ref[...], xq)

    logits = jax.lax.dot_general(qt, kt_ref[...], _DN_H,
                                 preferred_element_type=jnp.float32,
                                 precision=_PREC) * (1.0 / sqrt(_H))
    col = jax.lax.broadcasted_iota(jnp.int32, (_R, _NP), 1)
    neg = jnp.float32(-jnp.inf)
    logits = jnp.where(col < _N, logits, neg)

    g = logits
    for w in (1024, 512, 256, 128, 64, 32):
        g = jnp.maximum(g[:, :w], g[:, w:2 * w])
    m1 = jnp.max(g, axis=-1, keepdims=True)
    lo = jnp.min(g, axis=-1, keepdims=True)
    hi = m1
    for _ in range(_KBIN):
        mid = 0.5 * (lo + hi)
        cnt = jnp.sum(jnp.where(logits >= mid, 1.0, 0.0), axis=-1,
                      keepdims=True)
        pred = cnt >= float(_TOPK)
        lo = jnp.where(pred, mid, lo)
        hi = jnp.where(pred, hi, mid)

    p = jnp.where(logits >= lo, jnp.exp((logits - m1) * (1.0 / _TAU)), 0.0)
    propa = jax.lax.dot_general(xa, p, _DN_J,
                                preferred_element_type=jnp.float32,
                                precision=_PREC)
    prop = propa[0:_T]
    s_row = propa[_T:_T + 1]
    o_ref[0] = xq + _ALPHA * (prop * (1.0 / s_row))


def kernel(history_flow, Wq, Wk):
    return pl.pallas_call(
        _body,
        grid=(_B, _NP // _R),
        in_specs=[
            pl.BlockSpec((1, _T, _NP), lambda b, i: (b, 0, 0)),
            pl.BlockSpec((_H, _T), lambda b, i: (0, 0)),
            pl.BlockSpec((_H, _T), lambda b, i: (0, 0)),
        ],
        out_specs=pl.BlockSpec((1, _T, _R), lambda b, i: (b, 0, i)),
        out_shape=jax.ShapeDtypeStruct((_B, _T, _N), jnp.float32),
        scratch_shapes=[pltpu.VMEM((_H, _NP), jnp.float32),
                        pltpu.VMEM((_T + 8, _NP), jnp.float32)],
        compiler_params=pltpu.CompilerParams(
            dimension_semantics=("parallel", "arbitrary")),
    )(history_flow, Wq, Wk)


# speedup vs baseline: 1.0597x; 1.0597x over previous
from math import sqrt

import jax
import jax.numpy as jnp
from jax.experimental import pallas as pl
from jax.experimental.pallas import tpu as pltpu

_B, _T, _N = 8, 96, 2000
_H = 32
_TOPK = 20
_TAU = 0.5
_ALPHA = 0.15
_NP = 2048
_R = 1024
_KBIN = 10

_PREC = jax.lax.Precision.DEFAULT
_DN_T = (((1,), (0,)), ((), ()))
_DN_H = (((0,), (0,)), ((), ()))
_DN_J = (((1,), (1,)), ((), ()))


def _proj_norm(w, x, scale=1.0):
    p = jax.lax.dot_general(w, x, _DN_T, preferred_element_type=jnp.float32,
                            precision=_PREC)
    n = jnp.sqrt(jnp.sum(p * p, axis=0, keepdims=True))
    return p * (scale / jnp.maximum(n, 1e-12))


_QSCALE = 1.4426950408889634 / (sqrt(_H) * _TAU)


def _body(x_ref, wq_ref, wk_ref, o_ref, kt_ref, xm_ref):
    i = pl.program_id(1)

    @pl.when(i == 0)
    def _():
        colx = jax.lax.broadcasted_iota(jnp.int32, (_T, _NP), 1)
        xm = jnp.where(colx < _N, x_ref[0], 0.0)
        xm_ref[0:_T] = xm
        xm_ref[_T:] = jnp.ones((8, _NP), jnp.float32)
        kt_ref[0:_H] = _proj_norm(wk_ref[...], xm)
        colk = jax.lax.broadcasted_iota(jnp.int32, (1, _NP), 1)
        kt_ref[_H:_H + 1] = jnp.where(colk < _N, 0.0, -1e20)

    xa = xm_ref[...]
    x = xa[0:_T]

    xq = x_ref[0, :, pl.ds(i * _R, _R)]
    qt = _proj_norm(wq_ref[...], xq, _QSCALE)
    qt_a = jnp.concatenate([qt, jnp.ones((1, _R), jnp.float32)], axis=0)

    logits = jax.lax.dot_general(qt_a, kt_ref[...], _DN_H,
                                 preferred_element_type=jnp.float32,
                                 precision=_PREC)

    g = logits
    for w in (1024, 512, 256, 128, 64, 32):
        g = jnp.maximum(g[:, :w], g[:, w:2 * w])
    m1 = jnp.max(g, axis=-1, keepdims=True)
    lo = jnp.min(g, axis=-1, keepdims=True)
    hi = m1
    for _ in range(_KBIN):
        mid = 0.5 * (lo + hi)
        cnt = jnp.sum(jnp.where(logits >= mid, 1.0, 0.0), axis=-1,
                      keepdims=True)
        pred = cnt >= float(_TOPK)
        lo = jnp.where(pred, mid, lo)
        hi = jnp.where(pred, hi, mid)

    p = jnp.where(logits >= lo, jnp.exp2(logits), 0.0)
    propa = jax.lax.dot_general(xa, p, _DN_J,
                                preferred_element_type=jnp.float32,
                                precision=_PREC)
    prop = propa[0:_T]
    s_row = propa[_T:_T + 1]
    o_ref[0] = xq + _ALPHA * (prop * (1.0 / s_row))


def kernel(history_flow, Wq, Wk):
    return pl.pallas_call(
        _body,
        grid=(_B, _NP // _R),
        in_specs=[
            pl.BlockSpec((1, _T, _NP), lambda b, i: (b, 0, 0)),
            pl.BlockSpec((_H, _T), lambda b, i: (0, 0)),
            pl.BlockSpec((_H, _T), lambda b, i: (0, 0)),
        ],
        out_specs=pl.BlockSpec((1, _T, _R), lambda b, i: (b, 0, i)),
        out_shape=jax.ShapeDtypeStruct((_B, _T, _N), jnp.float32),
        scratch_shapes=[pltpu.VMEM((_H + 1, _NP), jnp.float32),
                        pltpu.VMEM((_T + 8, _NP), jnp.float32)],
        compiler_params=pltpu.CompilerParams(
            dimension_semantics=("parallel", "arbitrary")),
    )(history_flow, Wq, Wk)
